# baseline (speedup 1.0000x reference)
import functools

import jax
import jax.numpy as jnp
from jax import lax
from jax.experimental import pallas as pl
from jax.experimental.pallas import tpu as pltpu
from jax.experimental.pallas import tpu_sc as plsc

N_VOCAB = 1000000
D_MODEL = 64
BATCH = 4096
SEQ = 200

_NC = 2
_NS = 16
_NW = _NC * _NS

_B = BATCH * SEQ
_CH = 128
_BPW = _B // _NW
_NCHUNK = _BPW // _CH

_CB = 384
_NFULL = N_VOCAB // _CB
_TAIL = N_VOCAB - _NFULL * _CB
_TROWS = N_VOCAB // 2


def _cols_to_rows(src, dst, ncols):
    iota = lax.iota(jnp.int32, 16)

    def row(d, dv):
        for g in range(ncols // 16):
            j = g * 16 + iota
            vals = src[d, pl.ds(g * 16, 16)]
            plsc.store_scatter(dst, [j >> 1, (j & 1) * D_MODEL + dv], vals)
        return dv + 1

    lax.fori_loop(0, D_MODEL, row, jnp.zeros((16,), jnp.int32))


@functools.partial(
    pl.kernel,
    out_type=jax.ShapeDtypeStruct((_TROWS, 2 * D_MODEL), jnp.float32),
    mesh=plsc.VectorSubcoreMesh(core_axis_name="c", subcore_axis_name="s"),
    compiler_params=pltpu.CompilerParams(needs_layout_passes=False),
    scratch_types=[
        pltpu.VMEM((_CB // 128, D_MODEL, 128), jnp.float32),
        pltpu.VMEM((_CB // 2, 2 * D_MODEL), jnp.float32),
        pltpu.VMEM((D_MODEL, _TAIL), jnp.float32),
        pltpu.VMEM((_TAIL // 2, 2 * D_MODEL), jnp.float32),
        pltpu.SemaphoreType.DMA,
    ],
)
def _sc_transpose(w_hbm, tab_hbm, in_v, out_v, tin_v, tout_v, sem):
    wid = lax.axis_index("s") * _NC + lax.axis_index("c")
    nk = 81 + jnp.where(wid < _NFULL - 81 * _NW, 1, 0)

    def chunk(k, _):
        c = wid + k * _NW
        for t in range(_CB // 128):
            pltpu.async_copy(
                w_hbm.at[:, pl.ds(c * _CB + t * 128, 128)], in_v.at[t],
                sem).wait()
        for t in range(_CB // 128):
            _cols_to_rows(in_v.at[t], out_v.at[pl.ds(t * 64, 64)], 128)
        pltpu.sync_copy(out_v, tab_hbm.at[pl.ds(c * (_CB // 2), _CB // 2)])
        return _

    lax.fori_loop(0, nk, chunk, None)

    @pl.when(wid == _NW - 1)
    def _tail():
        pltpu.async_copy(w_hbm.at[:, pl.ds(_NFULL * _CB, _TAIL)], tin_v,
                         sem).wait()
        _cols_to_rows(tin_v, tout_v, _TAIL)
        pltpu.sync_copy(tout_v,
                        tab_hbm.at[pl.ds(_NFULL * (_CB // 2), _TAIL // 2)])


@functools.partial(
    pl.kernel,
    out_type=jax.ShapeDtypeStruct((_B, D_MODEL), jnp.float32),
    mesh=plsc.VectorSubcoreMesh(core_axis_name="c", subcore_axis_name="s"),
    compiler_params=pltpu.CompilerParams(use_tc_tiling_on_sc=False),
    scratch_types=[
        pltpu.VMEM((_NCHUNK, _CH), jnp.int32),
        pltpu.VMEM((_CH, D_MODEL), jnp.float32),
        pltpu.SemaphoreType.DMA,
    ],
)
def _sc_gather(table_hbm, idx_hbm, out_hbm, idx_v, out_v, sem):
    wid = lax.axis_index("s") * _NC + lax.axis_index("c")
    base = wid * _BPW
    pltpu.sync_copy(idx_hbm.at[wid], idx_v)

    def chunk(c, _):
        pltpu.async_copy(table_hbm.at[idx_v.at[c]], out_v, sem).wait()
        pltpu.sync_copy(out_v, out_hbm.at[pl.ds(base + c * _CH, _CH)])
        return _

    lax.fori_loop(0, _NCHUNK, chunk, None)


def kernel(x, W_embed):
    table = _sc_transpose(W_embed).reshape(N_VOCAB, D_MODEL)
    idx = x.astype(jnp.int32).reshape(_NW, _NCHUNK, _CH)
    out = _sc_gather(table, idx)
    return out.reshape(BATCH, SEQ, D_MODEL)
